# baseline (speedup 1.0000x reference)
import jax
import jax.numpy as jnp
from jax import lax
from jax.experimental import pallas as pl
from jax.experimental.pallas import tpu as pltpu
from jax.experimental.pallas import tpu_sc as plsc

VOCAB = 1000000
EMBED_DIM = 64
BATCH = 4096
SEQ = 200

_NC = 2
_NS = 16
_NW = _NC * _NS
_BB = BATCH // _NW

_NTCOL = VOCAB // 128
_COLS_MAIN = 244


def _fill_rotbuf(rotbuf, iota):
    for d in range(16):
        rot = (iota + d) & 15
        for k in range(8):
            rotbuf[8 * d + k, pl.ds(0, 16)] = rot + 16 * k


def _detile(table_t_hbm, tail_hbm, out_hbm, srcA, srcB, dstA, dstB, rotbuf,
            sgA, sgB, swA, swB):
    wid = lax.axis_index("s") * _NC + lax.axis_index("c")
    src = (srcA, srcB)
    dst = (dstA, dstB)
    sem_g = (sgA, sgB)
    sem_w = (swA, swB)
    iota = lax.iota(jnp.int32, 16)
    _fill_rotbuf(rotbuf, iota)
    fvm = [iota + 16 * m for m in range(4)]

    def col(ci):
        return 32 * ci + wid

    def fire_load(j, ci):
        pltpu.async_copy(
            table_t_hbm.at[:, pl.ds(col(ci) * 128, 128)], src[j], sem_g[j])

    def drain_load(j, ci):
        pltpu.make_async_copy(
            table_t_hbm.at[:, pl.ds(col(ci) * 128, 128)], src[j],
            sem_g[j]).wait()

    def fire_store(j, ci):
        pltpu.async_copy(dst[j], out_hbm.at[pl.ds(col(ci) * 8192, 8192)],
                         sem_w[j])

    def wait_store(j, ci):
        pltpu.make_async_copy(
            dst[j], out_hbm.at[pl.ds(col(ci) * 8192, 8192)], sem_w[j]).wait()

    def transpose(j, kmax=8):
        def d_body(d, carry):
            for k in range(kmax):
                vlv = rotbuf[8 * d + k, pl.ds(0, 16)]
                vs6 = vlv * 64
                for m in range(4):
                    val = plsc.load_gather(src[j], [fvm[m], vlv])
                    plsc.store_scatter(dst[j], [vs6 + fvm[m]], val)
            return carry

        lax.fori_loop(0, 16, d_body, 0)

    fire_load(0, 0)

    def step(t, carry):
        for j in range(2):
            ci = 2 * t + j
            fire_load(1 - j, ci + 1)
            drain_load(j, ci)

            @pl.when(t >= 1)
            def _():
                wait_store(j, ci)

            transpose(j)
            fire_store(j, ci)
        return carry

    lax.fori_loop(0, _COLS_MAIN // 2 - 1, step, 0)

    fire_load(1, _COLS_MAIN - 1)
    drain_load(0, _COLS_MAIN - 2)
    wait_store(0, 0)
    transpose(0)
    fire_store(0, _COLS_MAIN - 2)
    drain_load(1, _COLS_MAIN - 1)
    wait_store(1, 0)
    transpose(1)
    fire_store(1, _COLS_MAIN - 1)
    wait_store(0, 0)
    wait_store(1, 0)

    @pl.when(wid < 4)
    def _():
        c = 7808 + wid
        pltpu.sync_copy(table_t_hbm.at[:, pl.ds(c * 128, 128)], src[0])
        transpose(0)
        pltpu.sync_copy(dst[0], out_hbm.at[pl.ds(c * 8192, 8192)])

    @pl.when(wid == 4)
    def _():
        pltpu.sync_copy(tail_hbm, dst[0].at[pl.ds(0, 4096)])
        pltpu.sync_copy(dst[0].at[pl.ds(0, 4096)],
                        out_hbm.at[pl.ds(7812 * 8192, 4096)])


def _gather_t(idx_hbm, table_hbm, out_hbm, idxw, rawsA, rawsB, outA, outB,
              rotbuf, sgA, sgB, swA, swB):
    wid = lax.axis_index("s") * _NC + lax.axis_index("c")
    raws = (rawsA, rawsB)
    out_t = (outA, outB)
    sem_g = (sgA, sgB)
    sem_w = (swA, swB)
    iota = lax.iota(jnp.int32, 16)
    _fill_rotbuf(rotbuf, iota)
    fvm = [iota + 16 * m for m in range(4)]
    fim = [(iota + 16 * m) >> 3 for m in range(4)]
    innb = (iota & 7) * 128

    pltpu.sync_copy(idx_hbm.at[:, pl.ds(wid * _BB, _BB)], idxw)

    def fire_gather(j, s):
        pltpu.async_copy(table_hbm.at[idxw.at[s]], raws[j], sem_g[j])

    def drain_gather(j, s):
        pltpu.make_async_copy(
            table_hbm.at[idxw.at[s]], raws[j], sem_g[j]).wait()

    def fire_wb(j, s):
        pltpu.async_copy(out_t[j], out_hbm.at[s, :, wid], sem_w[j])

    def wait_wb(j, s):
        pltpu.make_async_copy(
            out_t[j], out_hbm.at[s, :, wid], sem_w[j]).wait()

    def transpose(j):
        def d_body(d, carry):
            for k in range(8):
                tv = rotbuf[8 * d + k, pl.ds(0, 16)]
                inner = innb + tv
                for m in range(4):
                    val = plsc.load_gather(raws[j], [tv, fvm[m]])
                    plsc.store_scatter(out_t[j], [fim[m], inner], val)
            return carry

        lax.fori_loop(0, 16, d_body, 0)

    fire_gather(0, 0)

    def step(m, carry):
        for j in range(2):
            s = 2 * m + j
            fire_gather(1 - j, s + 1)
            drain_gather(j, s)

            @pl.when(m >= 1)
            def _():
                wait_wb(j, s)

            transpose(j)
            fire_wb(j, s)
        return carry

    lax.fori_loop(0, SEQ // 2 - 1, step, 0)

    s0 = SEQ - 2
    fire_gather(1, SEQ - 1)
    drain_gather(0, s0)
    wait_wb(0, s0)
    transpose(0)
    fire_wb(0, s0)
    drain_gather(1, s0 + 1)
    wait_wb(1, s0 + 1)
    transpose(1)
    fire_wb(1, s0 + 1)
    wait_wb(0, s0)
    wait_wb(1, s0 + 1)


@jax.jit
def kernel(indices, embed_table):
    mesh = plsc.VectorSubcoreMesh(core_axis_name="c", subcore_axis_name="s")
    table_flat = pl.kernel(
        _detile,
        mesh=mesh,
        out_type=jax.ShapeDtypeStruct((VOCAB * EMBED_DIM,), jnp.float32),
        scratch_types=[
            pltpu.VMEM((EMBED_DIM, 128), jnp.float32),
            pltpu.VMEM((EMBED_DIM, 128), jnp.float32),
            pltpu.VMEM((8192,), jnp.float32),
            pltpu.VMEM((8192,), jnp.float32),
            pltpu.VMEM((128, 16), jnp.int32),
            pltpu.SemaphoreType.DMA,
            pltpu.SemaphoreType.DMA,
            pltpu.SemaphoreType.DMA,
            pltpu.SemaphoreType.DMA,
        ],
        compiler_params=pltpu.CompilerParams(
            use_tc_tiling_on_sc=True, needs_layout_passes=False),
    )(embed_table.T, embed_table[VOCAB - 64:].reshape(64 * EMBED_DIM))
    table_lin = table_flat.reshape(VOCAB, EMBED_DIM)
    idx2d = indices.T.astype(jnp.int32)
    out4 = pl.kernel(
        _gather_t,
        mesh=mesh,
        out_type=jax.ShapeDtypeStruct(
            (SEQ, EMBED_DIM // 8, BATCH // 128, 8 * 128), jnp.float32),
        scratch_types=[
            pltpu.VMEM((SEQ, _BB), jnp.int32),
            pltpu.VMEM((_BB, EMBED_DIM), jnp.float32),
            pltpu.VMEM((_BB, EMBED_DIM), jnp.float32),
            pltpu.VMEM((EMBED_DIM // 8, 8 * 128), jnp.float32),
            pltpu.VMEM((EMBED_DIM // 8, 8 * 128), jnp.float32),
            pltpu.VMEM((128, 16), jnp.int32),
            pltpu.SemaphoreType.DMA,
            pltpu.SemaphoreType.DMA,
            pltpu.SemaphoreType.DMA,
            pltpu.SemaphoreType.DMA,
        ],
        compiler_params=pltpu.CompilerParams(
            use_tc_tiling_on_sc=False, needs_layout_passes=False),
    )(idx2d, table_lin)
    out5 = out4.reshape(SEQ, EMBED_DIM // 8, BATCH // 128, 8, 128)
    return out5.transpose(2, 4, 0, 1, 3).reshape(BATCH, SEQ, EMBED_DIM)


# speedup vs baseline: 1.1341x; 1.1341x over previous
import jax
import jax.numpy as jnp
from jax import lax
from jax.experimental import pallas as pl
from jax.experimental.pallas import tpu as pltpu
from jax.experimental.pallas import tpu_sc as plsc

VOCAB = 1000000
EMBED_DIM = 64
BATCH = 4096
SEQ = 200

_NC = 2
_NS = 16
_NW = _NC * _NS
_BB = BATCH // _NW

_NTCOL = VOCAB // 128
_COLS_MAIN = 244


def _fill_rotbuf(rotbuf, iota):
    for d in range(16):
        rot = (iota + d) & 15
        for k in range(8):
            rotbuf[8 * d + k, pl.ds(0, 16)] = rot + 16 * k


def _detile(table_t_hbm, tail_hbm, out_hbm, srcA, srcB, dstA, dstB, rotbuf,
            sgA, sgB, swA, swB):
    wid = lax.axis_index("s") * _NC + lax.axis_index("c")
    src = (srcA, srcB)
    dst = (dstA, dstB)
    sem_g = (sgA, sgB)
    sem_w = (swA, swB)
    iota = lax.iota(jnp.int32, 16)
    _fill_rotbuf(rotbuf, iota)
    fvm = [iota + 16 * m for m in range(4)]

    def fire_load(j, ci):
        c2 = 32 * ci + wid
        pltpu.async_copy(
            table_t_hbm.at[:, pl.ds(c2 * 256, 256)], src[j], sem_g[j])

    def drain_load(j, ci):
        c2 = 32 * ci + wid
        pltpu.make_async_copy(
            table_t_hbm.at[:, pl.ds(c2 * 256, 256)], src[j],
            sem_g[j]).wait()

    def fire_store(j, ci):
        c2 = 32 * ci + wid
        pltpu.async_copy(dst[j], out_hbm.at[pl.ds(c2 * 16384, 16384)],
                         sem_w[j])

    def wait_store(j, ci):
        c2 = 32 * ci + wid
        pltpu.make_async_copy(
            dst[j], out_hbm.at[pl.ds(c2 * 16384, 16384)], sem_w[j]).wait()

    def transpose(j):
        def d_body(d, carry):
            for k in range(8):
                vlv = rotbuf[8 * d + k, pl.ds(0, 16)]
                vs6 = vlv * 64
                for h in range(2):
                    for m in range(4):
                        val = plsc.load_gather(
                            src[j], [fvm[m], vlv + 128 * h])
                        plsc.store_scatter(
                            dst[j], [vs6 + (fvm[m] + 8192 * h)], val)
            return carry

        lax.fori_loop(0, 16, d_body, 0)

    _NCI = 122

    fire_load(0, 0)

    def step(t, carry):
        for j in range(2):
            ci = 2 * t + j
            fire_load(1 - j, ci + 1)
            drain_load(j, ci)

            @pl.when(t >= 1)
            def _():
                wait_store(j, ci)

            transpose(j)
            fire_store(j, ci)
        return carry

    lax.fori_loop(0, _NCI // 2 - 1, step, 0)

    fire_load(1, _NCI - 1)
    drain_load(0, _NCI - 2)
    wait_store(0, 0)
    transpose(0)
    fire_store(0, _NCI - 2)
    drain_load(1, _NCI - 1)
    wait_store(1, 0)
    transpose(1)
    fire_store(1, _NCI - 1)
    wait_store(0, 0)
    wait_store(1, 0)

    @pl.when(wid < 2)
    def _():
        c2 = 3904 + wid
        pltpu.sync_copy(table_t_hbm.at[:, pl.ds(c2 * 256, 256)], src[0])
        transpose(0)
        pltpu.sync_copy(dst[0], out_hbm.at[pl.ds(c2 * 16384, 16384)])

    @pl.when(wid == 4)
    def _():
        pltpu.sync_copy(tail_hbm, dst[0].at[pl.ds(0, 4096)])
        pltpu.sync_copy(dst[0].at[pl.ds(0, 4096)],
                        out_hbm.at[pl.ds(7812 * 8192, 4096)])


def _gather_t(idx_hbm, table_hbm, out_hbm, idxw, idx2a, idx2b,
              rawsA, rawsB, outA, outB, rotbuf, sgA, sgB, swA, swB):
    wid = lax.axis_index("s") * _NC + lax.axis_index("c")
    raws = (rawsA, rawsB)
    out_t = (outA, outB)
    sem_g = (sgA, sgB)
    sem_w = (swA, swB)
    iota = lax.iota(jnp.int32, 16)
    _fill_rotbuf(rotbuf, iota)
    fvm = [iota + 16 * m for m in range(4)]
    fim = [(iota + 16 * m) >> 3 for m in range(4)]
    innb = (iota & 7) * 128

    pltpu.sync_copy(idx_hbm.at[:, pl.ds(wid * _BB, _BB)], idxw)

    idx2 = (idx2a, idx2b)

    def prep_idx(j, c):
        for s in range(2):
            for k in range(8):
                idx2[j][pl.ds(128 * s + 16 * k, 16)] = \
                    idxw[2 * c + s, pl.ds(16 * k, 16)]

    def fire_gather(j):
        pltpu.async_copy(table_hbm.at[idx2[j]], raws[j], sem_g[j])

    def drain_gather(j):
        pltpu.make_async_copy(
            table_hbm.at[idx2[j]], raws[j], sem_g[j]).wait()

    def fire_wb(j, c):
        pltpu.async_copy(out_t[j], out_hbm.at[pl.ds(2 * c, 2), :, wid],
                         sem_w[j])

    def wait_wb(j, c):
        pltpu.make_async_copy(
            out_t[j], out_hbm.at[pl.ds(2 * c, 2), :, wid], sem_w[j]).wait()

    def transpose(j):
        bvs = [iota * 0 + b for b in range(2)]

        def d_body(d, carry):
            for k in range(8):
                tv = rotbuf[8 * d + k, pl.ds(0, 16)]
                inner = innb + tv
                for b in range(2):
                    for m in range(4):
                        val = plsc.load_gather(
                            raws[j], [tv + 128 * b, fvm[m]])
                        plsc.store_scatter(
                            out_t[j], [bvs[b], fim[m], inner], val)
            return carry

        lax.fori_loop(0, 16, d_body, 0)

    _NCH = SEQ // 2

    prep_idx(0, 0)
    fire_gather(0)

    def step(t, carry):
        for j in range(2):
            c = 2 * t + j
            prep_idx(1 - j, c + 1)
            fire_gather(1 - j)
            drain_gather(j)

            @pl.when(t >= 1)
            def _():
                wait_wb(j, c)

            transpose(j)
            fire_wb(j, c)
        return carry

    lax.fori_loop(0, _NCH // 2 - 1, step, 0)

    c0 = _NCH - 2
    prep_idx(1, _NCH - 1)
    fire_gather(1)
    drain_gather(0)
    wait_wb(0, c0)
    transpose(0)
    fire_wb(0, c0)
    drain_gather(1)
    wait_wb(1, c0 + 1)
    transpose(1)
    fire_wb(1, c0 + 1)
    wait_wb(0, c0)
    wait_wb(1, c0 + 1)


@jax.jit
def kernel(indices, embed_table):
    mesh = plsc.VectorSubcoreMesh(core_axis_name="c", subcore_axis_name="s")
    table_flat = pl.kernel(
        _detile,
        mesh=mesh,
        out_type=jax.ShapeDtypeStruct((VOCAB * EMBED_DIM,), jnp.float32),
        scratch_types=[
            pltpu.VMEM((EMBED_DIM, 256), jnp.float32),
            pltpu.VMEM((EMBED_DIM, 256), jnp.float32),
            pltpu.VMEM((16384,), jnp.float32),
            pltpu.VMEM((16384,), jnp.float32),
            pltpu.VMEM((128, 16), jnp.int32),
            pltpu.SemaphoreType.DMA,
            pltpu.SemaphoreType.DMA,
            pltpu.SemaphoreType.DMA,
            pltpu.SemaphoreType.DMA,
        ],
        compiler_params=pltpu.CompilerParams(
            use_tc_tiling_on_sc=True, needs_layout_passes=False),
    )(embed_table.T, embed_table[VOCAB - 64:].reshape(64 * EMBED_DIM))
    table_lin = table_flat.reshape(VOCAB, EMBED_DIM)
    idx2d = indices.T.astype(jnp.int32)
    out4 = pl.kernel(
        _gather_t,
        mesh=mesh,
        out_type=jax.ShapeDtypeStruct(
            (SEQ, EMBED_DIM // 8, BATCH // 128, 8 * 128), jnp.float32),
        scratch_types=[
            pltpu.VMEM((SEQ, _BB), jnp.int32),
            pltpu.VMEM((2 * _BB,), jnp.int32),
            pltpu.VMEM((2 * _BB,), jnp.int32),
            pltpu.VMEM((2 * _BB, EMBED_DIM), jnp.float32),
            pltpu.VMEM((2 * _BB, EMBED_DIM), jnp.float32),
            pltpu.VMEM((2, EMBED_DIM // 8, 8 * 128), jnp.float32),
            pltpu.VMEM((2, EMBED_DIM // 8, 8 * 128), jnp.float32),
            pltpu.VMEM((128, 16), jnp.int32),
            pltpu.SemaphoreType.DMA,
            pltpu.SemaphoreType.DMA,
            pltpu.SemaphoreType.DMA,
            pltpu.SemaphoreType.DMA,
        ],
        compiler_params=pltpu.CompilerParams(
            use_tc_tiling_on_sc=False, needs_layout_passes=False),
    )(idx2d, table_lin)
    out5 = out4.reshape(SEQ, EMBED_DIM // 8, BATCH // 128, 8, 128)
    return out5.transpose(2, 4, 0, 1, 3).reshape(BATCH, SEQ, EMBED_DIM)
